# Initial kernel scaffold; baseline (speedup 1.0000x reference)
import jax
import jax.numpy as jnp
from jax.experimental import pallas as pl


def kernel(in_feat, edge_index, W_self1, W_neigh1, b1, W_self2, W_neigh2, b2):
    raise NotImplementedError("write your pallas kernel here")


# speedup vs baseline: 2.6897x; 2.6897x over previous
import functools

import jax
import jax.numpy as jnp
from jax import lax
from jax.experimental import pallas as pl
from jax.experimental.pallas import tpu as pltpu
from jax.experimental.pallas import tpu_sc as plsc

N_NODES = 10000
N_EDGES = 160000
D_IN = 256
D_HALF = 128

NTILES = 16
NCORES = 2
R = 10240
ROWS_PER_TILE = R // NTILES
ABSORB = N_NODES
CH = 128
NCH = 80
E_PAD = NTILES * NCH * CH


def _sc_agg_body(table, srcs, dsts, agg_out, src_v, dst_v, rows_v, agg_s):
    c = lax.axis_index("c")
    s = lax.axis_index("s")
    base = s * ROWS_PER_TILE

    def _zrow(k, _):
        rows_v[k // 8, pl.ds((k % 8) * 16, 16)] = jnp.zeros((16,), jnp.float32)
        return 0

    lax.fori_loop(0, CH * 8, _zrow, 0)
    for k in range(ROWS_PER_TILE // CH):
        pltpu.sync_copy(rows_v, agg_s.at[pl.ds(base + k * CH, CH)])

    plsc.subcore_barrier()

    def _chunk(j, _):
        pltpu.sync_copy(srcs.at[c, s, j], src_v)
        pltpu.sync_copy(dsts.at[s, j], dst_v)
        pltpu.sync_copy(table.at[src_v], rows_v)
        pltpu.sync_copy(rows_v, agg_s.at[dst_v], add=True)
        return 0

    lax.fori_loop(0, NCH, _chunk, 0)
    plsc.subcore_barrier()

    for k in range(ROWS_PER_TILE // CH):
        sl = pl.ds(base + k * CH, CH)
        pltpu.sync_copy(agg_s.at[sl], rows_v)
        pltpu.sync_copy(rows_v, agg_out.at[c, sl])


def _make_sc_agg():
    mesh = plsc.VectorSubcoreMesh(core_axis_name="c", subcore_axis_name="s")
    return pl.kernel(
        _sc_agg_body,
        out_type=jax.ShapeDtypeStruct((NCORES, R, D_HALF), jnp.float32),
        mesh=mesh,
        scratch_types=[
            pltpu.VMEM((CH,), jnp.int32),
            pltpu.VMEM((CH,), jnp.int32),
            pltpu.VMEM((CH, D_HALF), jnp.float32),
            pltpu.VMEM_SHARED((R, D_HALF), jnp.float32),
        ],
    )


def _sc_deg_body(dsts, deg_out, dst_v, ones_v, deg_s):
    c = lax.axis_index("c")
    s = lax.axis_index("s")
    base = s * ROWS_PER_TILE

    def _fill(v):
        def _row(k, _):
            ones_v[k // 8, pl.ds((k % 8) * 16, 16)] = jnp.full((16,), v, jnp.float32)
            return 0

        lax.fori_loop(0, CH * 8, _row, 0)

    _fill(0.0)
    for k in range(ROWS_PER_TILE // CH):
        pltpu.sync_copy(ones_v, deg_s.at[pl.ds(base + k * CH, CH)])
    _fill(1.0)
    plsc.subcore_barrier()

    def _chunk(j, _):
        pltpu.sync_copy(dsts.at[s, c * (NCH // 2) + j], dst_v)
        pltpu.sync_copy(ones_v, deg_s.at[dst_v], add=True)
        return 0

    lax.fori_loop(0, NCH // 2, _chunk, 0)
    plsc.subcore_barrier()

    for k in range(ROWS_PER_TILE // CH):
        sl = pl.ds(base + k * CH, CH)
        pltpu.sync_copy(deg_s.at[sl], ones_v)
        pltpu.sync_copy(ones_v, deg_out.at[c, sl])


def _make_sc_deg():
    mesh = plsc.VectorSubcoreMesh(core_axis_name="c", subcore_axis_name="s")
    return pl.kernel(
        _sc_deg_body,
        out_type=jax.ShapeDtypeStruct((NCORES, R, D_HALF), jnp.float32),
        mesh=mesh,
        scratch_types=[
            pltpu.VMEM((CH,), jnp.int32),
            pltpu.VMEM((CH, D_HALF), jnp.float32),
            pltpu.VMEM_SHARED((R, D_HALF), jnp.float32),
        ],
    )


def _tc_body(relu, split_out, x_ref, a_ref, deg_ref, ws_ref, wn_ref, b_ref, o_ref):
    deg = deg_ref[0][:, 0:1] + deg_ref[1][:, 0:1]
    inv = 1.0 / jnp.maximum(deg, 1.0)
    y = (
        jnp.dot(x_ref[0], ws_ref[0:128, :], preferred_element_type=jnp.float32)
        + jnp.dot(x_ref[1], ws_ref[128:256, :], preferred_element_type=jnp.float32)
        + jnp.dot(a_ref[0] * inv, wn_ref[0:128, :], preferred_element_type=jnp.float32)
        + jnp.dot(a_ref[1] * inv, wn_ref[128:256, :], preferred_element_type=jnp.float32)
        + b_ref[0:1, :]
    )
    if relu:
        y = jnp.maximum(y, 0.0)
    if split_out:
        o_ref[0] = y[:, 0:128]
        o_ref[1] = y[:, 128:256]
    else:
        o_ref[...] = y


def _make_tc_layer(relu, split_out, bn=2000):
    ngrid = N_NODES // bn
    if split_out:
        out_shape = jax.ShapeDtypeStruct((NCORES, R, D_HALF), jnp.float32)
        out_spec = pl.BlockSpec((NCORES, bn, D_HALF), lambda i: (0, i, 0))
    else:
        out_shape = jax.ShapeDtypeStruct((N_NODES, D_IN), jnp.float32)
        out_spec = pl.BlockSpec((bn, D_IN), lambda i: (i, 0))
    return pl.pallas_call(
        functools.partial(_tc_body, relu, split_out),
        grid=(ngrid,),
        in_specs=[
            pl.BlockSpec((NCORES, bn, D_HALF), lambda i: (0, i, 0)),
            pl.BlockSpec((NCORES, bn, D_HALF), lambda i: (0, i, 0)),
            pl.BlockSpec((NCORES, bn, D_HALF), lambda i: (0, i, 0)),
            pl.BlockSpec((D_IN, D_IN), lambda i: (0, 0)),
            pl.BlockSpec((D_IN, D_IN), lambda i: (0, 0)),
            pl.BlockSpec((1, D_IN), lambda i: (0, 0)),
        ],
        out_specs=out_spec,
        out_shape=out_shape,
    )


def kernel(in_feat, edge_index, W_self1, W_neigh1, b1, W_self2, W_neigh2, b2):
    ei = edge_index.astype(jnp.int32)
    src = jnp.concatenate(
        [ei[0], jnp.zeros((E_PAD - N_EDGES,), jnp.int32)])
    dst = jnp.concatenate(
        [ei[1], jnp.full((E_PAD - N_EDGES,), ABSORB, jnp.int32)])
    srcs = jnp.stack([src, src + R]).reshape(NCORES, NTILES, NCH, CH)
    dsts = dst.reshape(NTILES, NCH, CH)

    xs = jnp.transpose(in_feat.reshape(N_NODES, NCORES, D_HALF), (1, 0, 2))
    xs = jnp.pad(xs, ((0, 0), (0, R - N_NODES), (0, 0)))

    deg16 = _make_sc_deg()(dsts)
    agg1 = _make_sc_agg()(xs.reshape(NCORES * R, D_HALF), srcs, dsts)
    h = _make_tc_layer(True, True)(
        xs, agg1, deg16, W_self1, W_neigh1, b1.reshape(1, D_IN))
    agg2 = _make_sc_agg()(h.reshape(NCORES * R, D_HALF), srcs, dsts)
    return _make_tc_layer(False, False)(
        h, agg2, deg16, W_self2, W_neigh2, b2.reshape(1, D_IN))


# speedup vs baseline: 3.4292x; 1.2749x over previous
import functools

import jax
import jax.numpy as jnp
from jax import lax
from jax.experimental import pallas as pl
from jax.experimental.pallas import tpu as pltpu
from jax.experimental.pallas import tpu_sc as plsc

N_NODES = 10000
N_EDGES = 160000
D_IN = 256
D_HALF = 128

NTILES = 16
NCORES = 2
R = 10240
ROWS_PER_TILE = R // NTILES
ABSORB = N_NODES
CH = 128
NCH = 80
E_PAD = NTILES * NCH * CH


def _sc_agg_body(table, srcs, dsts, agg_out,
                 src_a, dst_a, rows0, rows1, g0, g1, s0, s1, agg_s):
    c = lax.axis_index("c")
    s = lax.axis_index("s")
    base = s * ROWS_PER_TILE

    def _zrow(k, _):
        rows0[k // 8, pl.ds((k % 8) * 16, 16)] = jnp.zeros((16,), jnp.float32)
        return 0

    lax.fori_loop(0, CH * 8, _zrow, 0)
    for k in range(ROWS_PER_TILE // CH):
        pltpu.sync_copy(rows0, agg_s.at[pl.ds(base + k * CH, CH)])

    plsc.subcore_barrier()

    nhalf = NCH // 2
    nb2 = nhalf // 2
    for half in range(2):
        pltpu.sync_copy(srcs.at[c, s, pl.ds(half * nhalf, nhalf)], src_a)
        pltpu.sync_copy(dsts.at[s, pl.ds(half * nhalf, nhalf)], dst_a)
        pltpu.async_copy(table.at[src_a.at[0]], rows0, g0)

        def _pair(k, _):
            j = 2 * k
            pltpu.make_async_copy(table.at[src_a.at[j]], rows0, g0).wait()
            pltpu.async_copy(table.at[src_a.at[j + 1]], rows1, g1)
            pltpu.async_copy(rows0, agg_s.at[dst_a.at[j]], s0, add=True)
            pltpu.make_async_copy(table.at[src_a.at[j + 1]], rows1, g1).wait()
            pltpu.make_async_copy(rows0, agg_s.at[dst_a.at[j]], s0).wait()

            @pl.when(k < nb2 - 1)
            def _():
                pltpu.async_copy(table.at[src_a.at[j + 2]], rows0, g0)

            pltpu.async_copy(rows1, agg_s.at[dst_a.at[j + 1]], s1, add=True)
            pltpu.make_async_copy(rows1, agg_s.at[dst_a.at[j + 1]], s1).wait()
            return 0

        lax.fori_loop(0, nb2, _pair, 0)
    plsc.subcore_barrier()

    for k in range(ROWS_PER_TILE // CH):
        sl = pl.ds(base + k * CH, CH)
        pltpu.sync_copy(agg_s.at[sl], rows0)
        pltpu.sync_copy(rows0, agg_out.at[c, sl])


def _make_sc_agg():
    mesh = plsc.VectorSubcoreMesh(core_axis_name="c", subcore_axis_name="s")
    return pl.kernel(
        _sc_agg_body,
        out_type=jax.ShapeDtypeStruct((NCORES, R, D_HALF), jnp.float32),
        mesh=mesh,
        scratch_types=[
            pltpu.VMEM((NCH // 2, CH), jnp.int32),
            pltpu.VMEM((NCH // 2, CH), jnp.int32),
            pltpu.VMEM((CH, D_HALF), jnp.float32),
            pltpu.VMEM((CH, D_HALF), jnp.float32),
            pltpu.SemaphoreType.DMA,
            pltpu.SemaphoreType.DMA,
            pltpu.SemaphoreType.DMA,
            pltpu.SemaphoreType.DMA,
            pltpu.VMEM_SHARED((R, D_HALF), jnp.float32),
        ],
    )


def _sc_deg_body(dsts, deg_out, dst_v, ones_v, deg_s):
    c = lax.axis_index("c")
    s = lax.axis_index("s")
    base = s * ROWS_PER_TILE

    def _fill(v):
        def _row(k, _):
            ones_v[k // 8, pl.ds((k % 8) * 16, 16)] = jnp.full((16,), v, jnp.float32)
            return 0

        lax.fori_loop(0, CH * 8, _row, 0)

    _fill(0.0)
    for k in range(ROWS_PER_TILE // CH):
        pltpu.sync_copy(ones_v, deg_s.at[pl.ds(base + k * CH, CH)])
    _fill(1.0)
    plsc.subcore_barrier()

    def _chunk(j, _):
        pltpu.sync_copy(dsts.at[s, c * (NCH // 2) + j], dst_v)
        pltpu.sync_copy(ones_v, deg_s.at[dst_v], add=True)
        return 0

    lax.fori_loop(0, NCH // 2, _chunk, 0)
    plsc.subcore_barrier()

    for k in range(ROWS_PER_TILE // CH):
        sl = pl.ds(base + k * CH, CH)
        pltpu.sync_copy(deg_s.at[sl], ones_v)
        pltpu.sync_copy(ones_v, deg_out.at[c, sl])


def _make_sc_deg():
    mesh = plsc.VectorSubcoreMesh(core_axis_name="c", subcore_axis_name="s")
    return pl.kernel(
        _sc_deg_body,
        out_type=jax.ShapeDtypeStruct((NCORES, R, D_HALF), jnp.float32),
        mesh=mesh,
        scratch_types=[
            pltpu.VMEM((CH,), jnp.int32),
            pltpu.VMEM((CH, D_HALF), jnp.float32),
            pltpu.VMEM_SHARED((R, D_HALF), jnp.float32),
        ],
    )


def _tc_body(relu, split_out, x_ref, a_ref, deg_ref, ws_ref, wn_ref, b_ref, o_ref):
    deg = deg_ref[0][:, 0:1] + deg_ref[1][:, 0:1]
    inv = 1.0 / jnp.maximum(deg, 1.0)
    y = (
        jnp.dot(x_ref[0], ws_ref[0:128, :], preferred_element_type=jnp.float32)
        + jnp.dot(x_ref[1], ws_ref[128:256, :], preferred_element_type=jnp.float32)
        + jnp.dot(a_ref[0] * inv, wn_ref[0:128, :], preferred_element_type=jnp.float32)
        + jnp.dot(a_ref[1] * inv, wn_ref[128:256, :], preferred_element_type=jnp.float32)
        + b_ref[0:1, :]
    )
    if relu:
        y = jnp.maximum(y, 0.0)
    if split_out:
        o_ref[0] = y[:, 0:128]
        o_ref[1] = y[:, 128:256]
    else:
        o_ref[...] = y


def _make_tc_layer(relu, split_out, bn=2000):
    ngrid = N_NODES // bn
    if split_out:
        out_shape = jax.ShapeDtypeStruct((NCORES, R, D_HALF), jnp.float32)
        out_spec = pl.BlockSpec((NCORES, bn, D_HALF), lambda i: (0, i, 0))
    else:
        out_shape = jax.ShapeDtypeStruct((N_NODES, D_IN), jnp.float32)
        out_spec = pl.BlockSpec((bn, D_IN), lambda i: (i, 0))
    return pl.pallas_call(
        functools.partial(_tc_body, relu, split_out),
        grid=(ngrid,),
        in_specs=[
            pl.BlockSpec((NCORES, bn, D_HALF), lambda i: (0, i, 0)),
            pl.BlockSpec((NCORES, bn, D_HALF), lambda i: (0, i, 0)),
            pl.BlockSpec((NCORES, bn, D_HALF), lambda i: (0, i, 0)),
            pl.BlockSpec((D_IN, D_IN), lambda i: (0, 0)),
            pl.BlockSpec((D_IN, D_IN), lambda i: (0, 0)),
            pl.BlockSpec((1, D_IN), lambda i: (0, 0)),
        ],
        out_specs=out_spec,
        out_shape=out_shape,
    )


def kernel(in_feat, edge_index, W_self1, W_neigh1, b1, W_self2, W_neigh2, b2):
    ei = edge_index.astype(jnp.int32)
    src = jnp.concatenate(
        [ei[0], jnp.zeros((E_PAD - N_EDGES,), jnp.int32)])
    dst = jnp.concatenate(
        [ei[1], jnp.full((E_PAD - N_EDGES,), ABSORB, jnp.int32)])
    srcs = jnp.stack([src, src + R]).reshape(NCORES, NTILES, NCH, CH)
    dsts = dst.reshape(NTILES, NCH, CH)

    xs = jnp.transpose(in_feat.reshape(N_NODES, NCORES, D_HALF), (1, 0, 2))
    xs = jnp.pad(xs, ((0, 0), (0, R - N_NODES), (0, 0)))

    deg16 = _make_sc_deg()(dsts)
    agg1 = _make_sc_agg()(xs.reshape(NCORES * R, D_HALF), srcs, dsts)
    h = _make_tc_layer(True, True)(
        xs, agg1, deg16, W_self1, W_neigh1, b1.reshape(1, D_IN))
    agg2 = _make_sc_agg()(h.reshape(NCORES * R, D_HALF), srcs, dsts)
    return _make_tc_layer(False, False)(
        h, agg2, deg16, W_self2, W_neigh2, b2.reshape(1, D_IN))


# speedup vs baseline: 3.7206x; 1.0850x over previous
import functools

import jax
import jax.numpy as jnp
from jax import lax
from jax.experimental import pallas as pl
from jax.experimental.pallas import tpu as pltpu
from jax.experimental.pallas import tpu_sc as plsc

N_NODES = 10000
N_EDGES = 160000
D_IN = 256
D_HALF = 128

NTILES = 16
NCORES = 2
R = 10240
ROWS_PER_TILE = R // NTILES
ABSORB = N_NODES
CH = 80
NCH = 128
QC = 32
NBUF = 4
E_PAD = NTILES * NCH * CH
ROW_CH = 128


def _sc_agg_body(table, srcs, dsts, agg_out,
                 src_a, dst_a, rows0, rows1, rows2, rows3,
                 g0, g1, g2, g3, s0, s1, s2, s3, agg_s):
    c = lax.axis_index("c")
    s = lax.axis_index("s")
    base = s * ROWS_PER_TILE
    rows = (rows0, rows1, rows2, rows3)
    gs = (g0, g1, g2, g3)
    ss = (s0, s1, s2, s3)

    def _zrow(k, _):
        rows0[k // 8, pl.ds((k % 8) * 16, 16)] = jnp.zeros((16,), jnp.float32)
        return 0

    lax.fori_loop(0, CH * 8, _zrow, 0)
    for k in range(ROWS_PER_TILE // CH):
        pltpu.sync_copy(rows0, agg_s.at[pl.ds(base + k * CH, CH)])

    plsc.subcore_barrier()

    for q in range(NCH // QC):
        pltpu.sync_copy(srcs.at[c, s, pl.ds(q * QC, QC)], src_a)
        pltpu.sync_copy(dsts.at[s, pl.ds(q * QC, QC)], dst_a)
        for b in range(NBUF):
            pltpu.async_copy(table.at[src_a.at[b]], rows[b], gs[b])

        def _grp(k, _):
            jj = NBUF * k
            for b in range(NBUF):
                pltpu.make_async_copy(table.at[src_a.at[jj + b]], rows[b], gs[b]).wait()
                pltpu.async_copy(rows[b], agg_s.at[dst_a.at[jj + b]], s0, add=True)
                pltpu.make_async_copy(rows[b], agg_s.at[dst_a.at[jj + b]], s0).wait()

                def _issue(b=b, jj=jj):
                    pltpu.async_copy(table.at[src_a.at[jj + b + NBUF]], rows[b], gs[b])

                pl.when(k < QC // NBUF - 1)(_issue)
            return 0

        lax.fori_loop(0, QC // NBUF, _grp, 0)
    plsc.subcore_barrier()

    pltpu.sync_copy(agg_s.at[pl.ds(base, ROWS_PER_TILE)],
                    agg_out.at[c, pl.ds(base, ROWS_PER_TILE)])


def _make_sc_agg():
    mesh = plsc.VectorSubcoreMesh(core_axis_name="c", subcore_axis_name="s")
    return pl.kernel(
        _sc_agg_body,
        out_type=jax.ShapeDtypeStruct((NCORES, R, D_HALF), jnp.float32),
        mesh=mesh,
        scratch_types=[
            pltpu.VMEM((QC, CH), jnp.int32),
            pltpu.VMEM((QC, CH), jnp.int32),
            pltpu.VMEM((CH, D_HALF), jnp.float32),
            pltpu.VMEM((CH, D_HALF), jnp.float32),
            pltpu.VMEM((CH, D_HALF), jnp.float32),
            pltpu.VMEM((CH, D_HALF), jnp.float32),
            pltpu.SemaphoreType.DMA,
            pltpu.SemaphoreType.DMA,
            pltpu.SemaphoreType.DMA,
            pltpu.SemaphoreType.DMA,
            pltpu.SemaphoreType.DMA,
            pltpu.SemaphoreType.DMA,
            pltpu.SemaphoreType.DMA,
            pltpu.SemaphoreType.DMA,
            pltpu.VMEM_SHARED((R, D_HALF), jnp.float32),
        ],
    )


def _sc_deg_body(dsts, deg_out, dst_a, ones_v, sem, deg_s):
    c = lax.axis_index("c")
    s = lax.axis_index("s")
    base = s * ROWS_PER_TILE
    nch = NCH // 2

    def _fill(v):
        def _row(k, _):
            ones_v[k // 8, pl.ds((k % 8) * 16, 16)] = jnp.full((16,), v, jnp.float32)
            return 0

        lax.fori_loop(0, CH * 8, _row, 0)

    _fill(0.0)
    for k in range(ROWS_PER_TILE // CH):
        pltpu.sync_copy(ones_v, deg_s.at[pl.ds(base + k * CH, CH)])
    _fill(1.0)
    pltpu.sync_copy(dsts.at[s, pl.ds(c * nch, nch)], dst_a)
    plsc.subcore_barrier()

    def _chunk(j, _):
        pltpu.async_copy(ones_v, deg_s.at[dst_a.at[j]], sem, add=True)
        pltpu.make_async_copy(ones_v, deg_s.at[dst_a.at[j]], sem).wait()
        return 0

    lax.fori_loop(0, nch, _chunk, 0)
    plsc.subcore_barrier()

    pltpu.sync_copy(deg_s.at[pl.ds(base, ROWS_PER_TILE)],
                    deg_out.at[c, pl.ds(base, ROWS_PER_TILE)])


def _make_sc_deg():
    mesh = plsc.VectorSubcoreMesh(core_axis_name="c", subcore_axis_name="s")
    return pl.kernel(
        _sc_deg_body,
        out_type=jax.ShapeDtypeStruct((NCORES, R, D_HALF), jnp.float32),
        mesh=mesh,
        scratch_types=[
            pltpu.VMEM((NCH // 2, CH), jnp.int32),
            pltpu.VMEM((CH, D_HALF), jnp.float32),
            pltpu.SemaphoreType.DMA,
            pltpu.VMEM_SHARED((R, D_HALF), jnp.float32),
        ],
    )


def _tc_body(relu, split_out, x_ref, a_ref, deg_ref, ws_ref, wn_ref, b_ref, o_ref):
    deg = deg_ref[0][:, 0:1] + deg_ref[1][:, 0:1]
    inv = 1.0 / jnp.maximum(deg, 1.0)
    y = (
        jnp.dot(x_ref[0], ws_ref[0:128, :], preferred_element_type=jnp.float32)
        + jnp.dot(x_ref[1], ws_ref[128:256, :], preferred_element_type=jnp.float32)
        + jnp.dot(a_ref[0] * inv, wn_ref[0:128, :], preferred_element_type=jnp.float32)
        + jnp.dot(a_ref[1] * inv, wn_ref[128:256, :], preferred_element_type=jnp.float32)
        + b_ref[0:1, :]
    )
    if relu:
        y = jnp.maximum(y, 0.0)
    if split_out:
        o_ref[0] = y[:, 0:128]
        o_ref[1] = y[:, 128:256]
    else:
        o_ref[...] = y


def _make_tc_layer(relu, split_out, bn=2000):
    ngrid = N_NODES // bn
    if split_out:
        out_shape = jax.ShapeDtypeStruct((NCORES, R, D_HALF), jnp.float32)
        out_spec = pl.BlockSpec((NCORES, bn, D_HALF), lambda i: (0, i, 0))
    else:
        out_shape = jax.ShapeDtypeStruct((N_NODES, D_IN), jnp.float32)
        out_spec = pl.BlockSpec((bn, D_IN), lambda i: (i, 0))
    return pl.pallas_call(
        functools.partial(_tc_body, relu, split_out),
        grid=(ngrid,),
        in_specs=[
            pl.BlockSpec((NCORES, bn, D_HALF), lambda i: (0, i, 0)),
            pl.BlockSpec((NCORES, bn, D_HALF), lambda i: (0, i, 0)),
            pl.BlockSpec((NCORES, bn, D_HALF), lambda i: (0, i, 0)),
            pl.BlockSpec((D_IN, D_IN), lambda i: (0, 0)),
            pl.BlockSpec((D_IN, D_IN), lambda i: (0, 0)),
            pl.BlockSpec((1, D_IN), lambda i: (0, 0)),
        ],
        out_specs=out_spec,
        out_shape=out_shape,
    )


def kernel(in_feat, edge_index, W_self1, W_neigh1, b1, W_self2, W_neigh2, b2):
    ei = edge_index.astype(jnp.int32)
    src = jnp.concatenate(
        [ei[0], jnp.zeros((E_PAD - N_EDGES,), jnp.int32)])
    dst = jnp.concatenate(
        [ei[1], jnp.full((E_PAD - N_EDGES,), ABSORB, jnp.int32)])
    srcs = jnp.stack([src, src + R]).reshape(NCORES, NTILES, NCH, CH)
    dsts = dst.reshape(NTILES, NCH, CH)

    xs = jnp.transpose(in_feat.reshape(N_NODES, NCORES, D_HALF), (1, 0, 2))
    xs = jnp.pad(xs, ((0, 0), (0, R - N_NODES), (0, 0)))

    deg16 = _make_sc_deg()(dsts)
    agg1 = _make_sc_agg()(xs.reshape(NCORES * R, D_HALF), srcs, dsts)
    h = _make_tc_layer(True, True)(
        xs, agg1, deg16, W_self1, W_neigh1, b1.reshape(1, D_IN))
    agg2 = _make_sc_agg()(h.reshape(NCORES * R, D_HALF), srcs, dsts)
    return _make_tc_layer(False, False)(
        h, agg2, deg16, W_self2, W_neigh2, b2.reshape(1, D_IN))


# speedup vs baseline: 3.9028x; 1.0490x over previous
import functools

import jax
import jax.numpy as jnp
from jax import lax
from jax.experimental import pallas as pl
from jax.experimental.pallas import tpu as pltpu
from jax.experimental.pallas import tpu_sc as plsc

N_NODES = 10000
N_EDGES = 160000
D_IN = 256
D_HALF = 128

NTILES = 16
NCORES = 2
R = 10240
ROWS_PER_TILE = R // NTILES
ABSORB = N_NODES
CH = 80
NCH = 128
QC = 32
NBUF = 4
E_PAD = NTILES * NCH * CH
ROW_CH = 128


def _sc_agg_body(table, srcs, dsts, agg_out,
                 src_a, dst_a, rows0, rows1, rows2, rows3,
                 g0, g1, g2, g3, s0, s1, s2, s3, agg_s):
    c = lax.axis_index("c")
    s = lax.axis_index("s")
    base = s * ROWS_PER_TILE
    rows = (rows0, rows1, rows2, rows3)
    gs = (g0, g1, g2, g3)
    ss = (s0, s1, s2, s3)

    def _zrow(k, _):
        rows0[k // 8, pl.ds((k % 8) * 16, 16)] = jnp.zeros((16,), jnp.float32)
        return 0

    lax.fori_loop(0, CH * 8, _zrow, 0)
    for k in range(ROWS_PER_TILE // CH):
        pltpu.sync_copy(rows0, agg_s.at[pl.ds(base + k * CH, CH)])

    plsc.subcore_barrier()

    for q in range(NCH // QC):
        pltpu.sync_copy(srcs.at[c, s, pl.ds(q * QC, QC)], src_a)
        pltpu.sync_copy(dsts.at[s, pl.ds(q * QC, QC)], dst_a)
        for b in range(NBUF):
            pltpu.async_copy(table.at[src_a.at[b]], rows[b], gs[b])

        def _grp(k, _):
            jj = NBUF * k
            for b in range(NBUF):
                pltpu.make_async_copy(table.at[src_a.at[jj + b]], rows[b], gs[b]).wait()
                pltpu.async_copy(rows[b], agg_s.at[dst_a.at[jj + b]], s0, add=True)
                pltpu.make_async_copy(rows[b], agg_s.at[dst_a.at[jj + b]], s0).wait()

                def _issue(b=b, jj=jj):
                    pltpu.async_copy(table.at[src_a.at[jj + b + NBUF]], rows[b], gs[b])

                pl.when(k < QC // NBUF - 1)(_issue)
            return 0

        lax.fori_loop(0, QC // NBUF, _grp, 0)
    plsc.subcore_barrier()

    pltpu.sync_copy(agg_s.at[pl.ds(base, ROWS_PER_TILE)],
                    agg_out.at[c, pl.ds(base, ROWS_PER_TILE)])


def _make_sc_agg():
    mesh = plsc.VectorSubcoreMesh(core_axis_name="c", subcore_axis_name="s")
    return pl.kernel(
        _sc_agg_body,
        out_type=jax.ShapeDtypeStruct((NCORES, R, D_HALF), jnp.float32),
        mesh=mesh,
        scratch_types=[
            pltpu.VMEM((QC, CH), jnp.int32),
            pltpu.VMEM((QC, CH), jnp.int32),
            pltpu.VMEM((CH, D_HALF), jnp.float32),
            pltpu.VMEM((CH, D_HALF), jnp.float32),
            pltpu.VMEM((CH, D_HALF), jnp.float32),
            pltpu.VMEM((CH, D_HALF), jnp.float32),
            pltpu.SemaphoreType.DMA,
            pltpu.SemaphoreType.DMA,
            pltpu.SemaphoreType.DMA,
            pltpu.SemaphoreType.DMA,
            pltpu.SemaphoreType.DMA,
            pltpu.SemaphoreType.DMA,
            pltpu.SemaphoreType.DMA,
            pltpu.SemaphoreType.DMA,
            pltpu.VMEM_SHARED((R, D_HALF), jnp.float32),
        ],
    )


def _sc_deg_body(dsts, deg_out, dst_a, ones_v, sem, deg_s):
    c = lax.axis_index("c")
    s = lax.axis_index("s")
    base = s * ROWS_PER_TILE
    nch = NCH // 2

    def _fill(v):
        def _row(k, _):
            ones_v[k // 8, pl.ds((k % 8) * 16, 16)] = jnp.full((16,), v, jnp.float32)
            return 0

        lax.fori_loop(0, CH * 8, _row, 0)

    _fill(0.0)
    for k in range(ROWS_PER_TILE // CH):
        pltpu.sync_copy(ones_v, deg_s.at[pl.ds(base + k * CH, CH)])
    _fill(1.0)
    pltpu.sync_copy(dsts.at[s, pl.ds(c * nch, nch)], dst_a)
    plsc.subcore_barrier()

    def _chunk(j, _):
        pltpu.async_copy(ones_v, deg_s.at[dst_a.at[j]], sem, add=True)
        pltpu.make_async_copy(ones_v, deg_s.at[dst_a.at[j]], sem).wait()
        return 0

    lax.fori_loop(0, nch, _chunk, 0)
    plsc.subcore_barrier()

    pltpu.sync_copy(deg_s.at[pl.ds(base, ROWS_PER_TILE)],
                    deg_out.at[c, pl.ds(base, ROWS_PER_TILE)])


def _make_sc_deg():
    mesh = plsc.VectorSubcoreMesh(core_axis_name="c", subcore_axis_name="s")
    return pl.kernel(
        _sc_deg_body,
        out_type=jax.ShapeDtypeStruct((NCORES, R, D_HALF), jnp.float32),
        mesh=mesh,
        scratch_types=[
            pltpu.VMEM((NCH // 2, CH), jnp.int32),
            pltpu.VMEM((CH, D_HALF), jnp.float32),
            pltpu.SemaphoreType.DMA,
            pltpu.VMEM_SHARED((R, D_HALF), jnp.float32),
        ],
    )


def _tc_body(relu, split_in, split_out, x_ref, a_ref, deg_ref, ws_ref, wn_ref, b_ref, o_ref):
    deg = deg_ref[0][:, 0:1] + deg_ref[1][:, 0:1]
    inv = 1.0 / jnp.maximum(deg, 1.0)
    if split_in:
        ys = (jnp.dot(x_ref[0], ws_ref[0:128, :], preferred_element_type=jnp.float32)
              + jnp.dot(x_ref[1], ws_ref[128:256, :], preferred_element_type=jnp.float32))
    else:
        ys = jnp.dot(x_ref[...], ws_ref[...], preferred_element_type=jnp.float32)
    y = (
        ys
        + jnp.dot(a_ref[0] * inv, wn_ref[0:128, :], preferred_element_type=jnp.float32)
        + jnp.dot(a_ref[1] * inv, wn_ref[128:256, :], preferred_element_type=jnp.float32)
        + b_ref[0:1, :]
    )
    if relu:
        y = jnp.maximum(y, 0.0)
    if split_out:
        o_ref[0] = y[:, 0:128]
        o_ref[1] = y[:, 128:256]
    else:
        o_ref[...] = y


def _make_tc_layer(relu, split_in, split_out, bn=2000):
    ngrid = N_NODES // bn
    if split_in:
        x_spec = pl.BlockSpec((NCORES, bn, D_HALF), lambda i: (0, i, 0))
    else:
        x_spec = pl.BlockSpec((bn, D_IN), lambda i: (i, 0))
    if split_out:
        out_shape = jax.ShapeDtypeStruct((NCORES, R, D_HALF), jnp.float32)
        out_spec = pl.BlockSpec((NCORES, bn, D_HALF), lambda i: (0, i, 0))
    else:
        out_shape = jax.ShapeDtypeStruct((N_NODES, D_IN), jnp.float32)
        out_spec = pl.BlockSpec((bn, D_IN), lambda i: (i, 0))
    return pl.pallas_call(
        functools.partial(_tc_body, relu, split_in, split_out),
        grid=(ngrid,),
        in_specs=[
            x_spec,
            pl.BlockSpec((NCORES, bn, D_HALF), lambda i: (0, i, 0)),
            pl.BlockSpec((NCORES, bn, D_HALF), lambda i: (0, i, 0)),
            pl.BlockSpec((D_IN, D_IN), lambda i: (0, 0)),
            pl.BlockSpec((D_IN, D_IN), lambda i: (0, 0)),
            pl.BlockSpec((1, D_IN), lambda i: (0, 0)),
        ],
        out_specs=out_spec,
        out_shape=out_shape,
    )


def kernel(in_feat, edge_index, W_self1, W_neigh1, b1, W_self2, W_neigh2, b2):
    ei = edge_index.astype(jnp.int32)
    src = jnp.concatenate(
        [ei[0], jnp.zeros((E_PAD - N_EDGES,), jnp.int32)])
    dst = jnp.concatenate(
        [ei[1], jnp.full((E_PAD - N_EDGES,), ABSORB, jnp.int32)])
    srcs1 = jnp.stack([2 * src, 2 * src + 1]).reshape(NCORES, NTILES, NCH, CH)
    srcs2 = jnp.stack([src, src + R]).reshape(NCORES, NTILES, NCH, CH)
    dsts = dst.reshape(NTILES, NCH, CH)

    deg16 = _make_sc_deg()(dsts)
    agg1 = _make_sc_agg()(in_feat.reshape(2 * N_NODES, D_HALF), srcs1, dsts)
    h = _make_tc_layer(True, False, True)(
        in_feat, agg1, deg16, W_self1, W_neigh1, b1.reshape(1, D_IN))
    agg2 = _make_sc_agg()(h.reshape(NCORES * R, D_HALF), srcs2, dsts)
    return _make_tc_layer(False, True, False)(
        h, agg2, deg16, W_self2, W_neigh2, b2.reshape(1, D_IN))
